# baseline (speedup 1.0000x reference)
import functools

import jax
import jax.numpy as jnp
from jax import lax
from jax.experimental import pallas as pl
from jax.experimental.pallas import tpu as pltpu
from jax.experimental.pallas import tpu_sc as plsc

N = 10000
E = 320000
D_IN = 128
D_HID = 256
N_CLASSES = 40
W48 = 48

NC = 2
NS = 16
NW = NC * NS
NP = 10240
K = 128
EPT = NP
NCHUNK = EPT // K
EPAD = NW * NP - E
RPT = NP // NS

_MESH = plsc.VectorSubcoreMesh(
    core_axis_name="c", subcore_axis_name="s", num_cores=NC, num_subcores=NS
)
_SC_PARAMS = pltpu.CompilerParams(
    needs_layout_passes=False, use_tc_tiling_on_sc=False
)

BLK = 1024
GRID = NP // BLK
BLK3 = 1000
GRID3 = N // BLK3


@functools.partial(
    pl.kernel,
    out_type=jax.ShapeDtypeStruct((NW, NP), jnp.float32),
    mesh=_MESH,
    compiler_params=_SC_PARAMS,
    scratch_types=[
        pltpu.VMEM((EPT,), jnp.int32),
        pltpu.VMEM((NP,), jnp.float32),
    ],
)
def _sc_degree(dst_hbm, out_hbm, dsti_v, deg_v):
    cid = lax.axis_index("c")
    sid = lax.axis_index("s")
    wid = sid * NC + cid
    zeros16 = jnp.zeros((16,), jnp.float32)

    @pl.loop(0, NP // 16)
    def _zero(z):
        deg_v[pl.ds(z * 16, 16)] = zeros16

    pltpu.sync_copy(dst_hbm.at[wid], dsti_v)
    ones16 = jnp.ones((16,), jnp.float32)

    @pl.loop(0, EPT // 16)
    def _count(g):
        idx16 = dsti_v[pl.ds(g * 16, 16)]
        plsc.addupdate_scatter(deg_v, [idx16], ones16)

    pltpu.sync_copy(deg_v, out_hbm.at[wid])


@functools.partial(
    pl.kernel,
    out_type=jax.ShapeDtypeStruct((NC, NP, W48), jnp.float32),
    mesh=_MESH,
    compiler_params=_SC_PARAMS,
    scratch_types=[
        pltpu.MemorySpace.VMEM_SHARED((NP, W48), jnp.float32),
        pltpu.VMEM((NCHUNK, K), jnp.int32),
        pltpu.VMEM((NCHUNK, K), jnp.int32),
        [pltpu.VMEM((K, W48), jnp.float32)] * 4,
        pltpu.VMEM((RPT, W48), jnp.float32),
        [pltpu.SemaphoreType.DMA] * 4,
        [pltpu.SemaphoreType.DMA] * 4,
    ],
)
def _sc_propagate(table_hbm, src_hbm, dst_hbm, out_hbm,
                  accum, srcv, dstv, rows, zbuf, semg, sems):
    cid = lax.axis_index("c")
    sid = lax.axis_index("s")
    wid = sid * NC + cid
    zeros16 = jnp.zeros((16,), jnp.float32)

    @pl.loop(0, RPT)
    def _zrow(r):
        for c3 in range(W48 // 16):
            zbuf[r, pl.ds(c3 * 16, 16)] = zeros16

    pltpu.sync_copy(zbuf, accum.at[pl.ds(sid * RPT, RPT)])

    pltpu.sync_copy(src_hbm.at[wid], srcv)
    pltpu.sync_copy(dst_hbm.at[wid], dstv)
    plsc.subcore_barrier()

    def gather(j, u):
        return pltpu.async_copy(table_hbm.at[srcv.at[j]], rows[u], semg[u])

    def scat(j, u):
        return pltpu.async_copy(rows[u], accum.at[dstv.at[j]], sems[u],
                                add=True)

    gather(0, 0)
    gather(1, 1)

    @pl.loop(0, NCHUNK // 4)
    def _edges(q):
        for u in range(4):
            j = q * 4 + u
            pltpu.make_async_copy(table_hbm.at[srcv.at[j]], rows[u],
                                  semg[u]).wait()
            scat(j, u)
            un = (u + 2) % 4

            def wait_scat():
                pltpu.make_async_copy(rows[un], accum.at[dstv.at[j - 2]],
                                      sems[un]).wait()

            if u < 2:
                @pl.when(q > 0)
                def _():
                    wait_scat()

                gather(j + 2, un)
            else:
                wait_scat()

                @pl.when(q < NCHUNK // 4 - 1)
                def _():
                    gather(j + 2, un)

    for jt, u in ((NCHUNK - 2, (NCHUNK - 2) % 4), (NCHUNK - 1, (NCHUNK - 1) % 4)):
        pltpu.make_async_copy(rows[u], accum.at[dstv.at[jt]], sems[u]).wait()
    plsc.subcore_barrier()

    pltpu.sync_copy(accum.at[pl.ds(sid * RPT, RPT)], zbuf)
    pltpu.sync_copy(zbuf, out_hbm.at[cid, pl.ds(sid * RPT, RPT)])


def _colmask(nrows):
    return (lax.broadcasted_iota(jnp.int32, (nrows, W48), 1) == 40).astype(
        jnp.float32
    )


def _col40(a):
    return jnp.sum(jnp.where(_colmask(a.shape[0]) > 0, a, 0.0), axis=1,
                   keepdims=True)


def _tc1_body(x_ref, w1_ref, w2p_ref, degp_ref, y1_ref):
    w12 = jnp.dot(w1_ref[...], w2p_ref[...], preferred_element_type=jnp.float32)
    p = jnp.dot(x_ref[...], w12, preferred_element_type=jnp.float32)
    deg = jnp.sum(degp_ref[...], axis=0)[:, None] + 1.0
    dinv = lax.rsqrt(deg)
    y1_ref[...] = dinv * (p + _colmask(p.shape[0]))


def _tc2_body(acc0_ref, acc1_ref, y1_ref, y2_ref):
    y1 = y1_ref[...]
    dinv = _col40(y1)
    t = acc0_ref[...] + acc1_ref[...] + y1
    y2_ref[...] = (dinv * dinv) * t


def _tc3_body(acc0_ref, acc1_ref, y2_ref, y1_ref, w2p_ref, b1_ref, b2p_ref,
              out_ref):
    y2 = y2_ref[...]
    dinv = _col40(y1_ref[...])
    svec = _col40(y2) / dinv
    bw = jnp.dot(b1_ref[...], w2p_ref[...], preferred_element_type=jnp.float32)
    o = dinv * (acc0_ref[...] + acc1_ref[...] + y2)
    o = o + svec * bw + b2p_ref[...]
    out_ref[...] = o[:, :N_CLASSES]


def kernel(x, edge_index, W1, b1, W2, b2):
    padr = N + jnp.arange(EPAD, dtype=jnp.int32) % (NP - N)
    ei = jnp.concatenate([edge_index, jnp.stack([padr, padr])], axis=1)
    src = ei[0].reshape(NW, NCHUNK, K)
    dst = ei[1].reshape(NW, NCHUNK, K)
    dst_flat = ei[1].reshape(NW, EPT)
    xp = jnp.pad(x, ((0, NP - N), (0, 0)))

    w2p = jnp.pad(W2, ((0, 0), (0, W48 - N_CLASSES)))
    b1r = b1[None, :]
    b2p = jnp.pad(b2, (0, W48 - N_CLASSES))[None, :]

    degp = _sc_degree(dst_flat)

    y1 = pl.pallas_call(
        _tc1_body,
        grid=(GRID,),
        in_specs=[
            pl.BlockSpec((BLK, D_IN), lambda i: (i, 0)),
            pl.BlockSpec((D_IN, D_HID), lambda i: (0, 0)),
            pl.BlockSpec((D_HID, W48), lambda i: (0, 0)),
            pl.BlockSpec((NW, BLK), lambda i: (0, i)),
        ],
        out_specs=pl.BlockSpec((BLK, W48), lambda i: (i, 0)),
        out_shape=jax.ShapeDtypeStruct((NP, W48), jnp.float32),
    )(xp, W1, w2p, degp)

    accb = _sc_propagate(y1, src, dst)
    acc0b, acc1b = accb[0], accb[1]

    y2 = pl.pallas_call(
        _tc2_body,
        grid=(GRID,),
        in_specs=[
            pl.BlockSpec((BLK, W48), lambda i: (i, 0)),
            pl.BlockSpec((BLK, W48), lambda i: (i, 0)),
            pl.BlockSpec((BLK, W48), lambda i: (i, 0)),
        ],
        out_specs=pl.BlockSpec((BLK, W48), lambda i: (i, 0)),
        out_shape=jax.ShapeDtypeStruct((NP, W48), jnp.float32),
    )(acc0b, acc1b, y1)

    accc = _sc_propagate(y2, src, dst)
    acc0c, acc1c = accc[0], accc[1]

    out = pl.pallas_call(
        _tc3_body,
        grid=(GRID,),
        in_specs=[
            pl.BlockSpec((BLK, W48), lambda i: (i, 0)),
            pl.BlockSpec((BLK, W48), lambda i: (i, 0)),
            pl.BlockSpec((BLK, W48), lambda i: (i, 0)),
            pl.BlockSpec((BLK, W48), lambda i: (i, 0)),
            pl.BlockSpec((D_HID, W48), lambda i: (0, 0)),
            pl.BlockSpec((1, D_HID), lambda i: (0, 0)),
            pl.BlockSpec((1, W48), lambda i: (0, 0)),
        ],
        out_specs=pl.BlockSpec((BLK, N_CLASSES), lambda i: (i, 0)),
        out_shape=jax.ShapeDtypeStruct((NP, N_CLASSES), jnp.float32),
    )(acc0c, acc1c, y2, y1, w2p, b1r, b2p)

    return out[:N]


# speedup vs baseline: 1.0675x; 1.0675x over previous
import functools

import jax
import jax.numpy as jnp
from jax import lax
from jax.experimental import pallas as pl
from jax.experimental.pallas import tpu as pltpu
from jax.experimental.pallas import tpu_sc as plsc

N = 10000
E = 320000
D_IN = 128
D_HID = 256
N_CLASSES = 40
W48 = 48

NC = 2
NS = 16
NW = NC * NS
NP = 10240
K = 128
EPT = NP
NCHUNK = EPT // K
EPAD = NW * NP - E
RPT = NP // NS

_MESH = plsc.VectorSubcoreMesh(
    core_axis_name="c", subcore_axis_name="s", num_cores=NC, num_subcores=NS
)
_SC_PARAMS = pltpu.CompilerParams(
    needs_layout_passes=False, use_tc_tiling_on_sc=False
)

BLK = 1024
GRID = NP // BLK
BLK3 = 1000
GRID3 = N // BLK3


@functools.partial(
    pl.kernel,
    out_type=jax.ShapeDtypeStruct((NW, NP), jnp.float32),
    mesh=_MESH,
    compiler_params=_SC_PARAMS,
    scratch_types=[
        pltpu.VMEM((EPT,), jnp.int32),
        pltpu.VMEM((NP,), jnp.float32),
    ],
)
def _sc_degree(dst_hbm, out_hbm, dsti_v, deg_v):
    cid = lax.axis_index("c")
    sid = lax.axis_index("s")
    wid = sid * NC + cid
    zeros16 = jnp.zeros((16,), jnp.float32)

    @pl.loop(0, NP // 16)
    def _zero(z):
        deg_v[pl.ds(z * 16, 16)] = zeros16

    pltpu.sync_copy(dst_hbm.at[wid], dsti_v)
    ones16 = jnp.ones((16,), jnp.float32)

    @pl.loop(0, EPT // 16)
    def _count(g):
        idx16 = dsti_v[pl.ds(g * 16, 16)]
        plsc.addupdate_scatter(deg_v, [idx16], ones16)

    pltpu.sync_copy(deg_v, out_hbm.at[wid])


@functools.partial(
    pl.kernel,
    out_type=jax.ShapeDtypeStruct((NC, NP, W48), jnp.float32),
    mesh=_MESH,
    compiler_params=_SC_PARAMS,
    scratch_types=[
        pltpu.MemorySpace.VMEM_SHARED((NP, W48), jnp.float32),
        pltpu.VMEM((NCHUNK, K), jnp.int32),
        pltpu.VMEM((NCHUNK, K), jnp.int32),
        [pltpu.VMEM((K, W48), jnp.float32)] * 4,
        pltpu.VMEM((RPT, W48), jnp.float32),
        [pltpu.SemaphoreType.DMA] * 4,
        [pltpu.SemaphoreType.DMA] * 4,
    ],
)
def _sc_propagate(table_hbm, src_hbm, dst_hbm, out_hbm,
                  accum, srcv, dstv, rows, zbuf, semg, sems):
    cid = lax.axis_index("c")
    sid = lax.axis_index("s")
    wid = sid * NC + cid
    zeros16 = jnp.zeros((16,), jnp.float32)

    @pl.loop(0, RPT)
    def _zrow(r):
        for c3 in range(W48 // 16):
            zbuf[r, pl.ds(c3 * 16, 16)] = zeros16

    pltpu.sync_copy(zbuf, accum.at[pl.ds(sid * RPT, RPT)])

    pltpu.sync_copy(src_hbm.at[wid], srcv)
    pltpu.sync_copy(dst_hbm.at[wid], dstv)
    plsc.subcore_barrier()

    def gather(j, u):
        return pltpu.async_copy(table_hbm.at[srcv.at[j]], rows[u], semg[u])

    def scat(j, u):
        return pltpu.async_copy(rows[u], accum.at[dstv.at[j]], sems[u],
                                add=True)

    gather(0, 0)
    gather(1, 1)

    @pl.loop(0, NCHUNK // 4)
    def _edges(q):
        for u in range(4):
            j = q * 4 + u
            pltpu.make_async_copy(table_hbm.at[srcv.at[j]], rows[u],
                                  semg[u]).wait()
            scat(j, u)
            un = (u + 2) % 4

            def wait_scat():
                pltpu.make_async_copy(rows[un], accum.at[dstv.at[j - 2]],
                                      sems[un]).wait()

            if u < 2:
                @pl.when(q > 0)
                def _():
                    wait_scat()

                gather(j + 2, un)
            else:
                wait_scat()

                @pl.when(q < NCHUNK // 4 - 1)
                def _():
                    gather(j + 2, un)

    for jt, u in ((NCHUNK - 2, (NCHUNK - 2) % 4), (NCHUNK - 1, (NCHUNK - 1) % 4)):
        pltpu.make_async_copy(rows[u], accum.at[dstv.at[jt]], sems[u]).wait()
    plsc.subcore_barrier()

    pltpu.sync_copy(accum.at[pl.ds(sid * RPT, RPT)], zbuf)
    pltpu.sync_copy(zbuf, out_hbm.at[cid, pl.ds(sid * RPT, RPT)])


def _colmask(nrows):
    return (lax.broadcasted_iota(jnp.int32, (nrows, W48), 1) == 40).astype(
        jnp.float32
    )


def _col40(a):
    return jnp.sum(jnp.where(_colmask(a.shape[0]) > 0, a, 0.0), axis=1,
                   keepdims=True)


def _tc1_body(x_ref, w1_ref, w2p_ref, degp_ref, y1_ref):
    w12 = jnp.dot(w1_ref[...], w2p_ref[...], preferred_element_type=jnp.float32)
    p = jnp.dot(x_ref[...], w12, preferred_element_type=jnp.float32)
    deg = jnp.sum(degp_ref[...], axis=0)[:, None] + 1.0
    dinv = lax.rsqrt(deg)
    y1_ref[...] = dinv * (p + _colmask(p.shape[0]))


def _tc2_body(acc_ref, y1_ref, y2_ref):
    y1 = y1_ref[...]
    dinv = _col40(y1)
    t = acc_ref[0] + acc_ref[1] + y1
    y2_ref[...] = (dinv * dinv) * t


def _tc3_body(acc_ref, y2_ref, y1_ref, w2p_ref, b1_ref, b2p_ref, out_ref):
    y2 = y2_ref[...]
    dinv = _col40(y1_ref[...])
    svec = _col40(y2) / dinv
    bw = jnp.dot(b1_ref[...], w2p_ref[...], preferred_element_type=jnp.float32)
    o = dinv * (acc_ref[0] + acc_ref[1] + y2)
    o = o + svec * bw + b2p_ref[...]
    out_ref[...] = o[:, :N_CLASSES]


def kernel(x, edge_index, W1, b1, W2, b2):
    padr = N + jnp.arange(EPAD, dtype=jnp.int32) % (NP - N)
    ei = jnp.concatenate([edge_index, jnp.stack([padr, padr])], axis=1)
    src = ei[0].reshape(NW, NCHUNK, K)
    dst = ei[1].reshape(NW, NCHUNK, K)
    dst_flat = ei[1].reshape(NW, EPT)
    xp = jnp.pad(x, ((0, NP - N), (0, 0)))

    w2p = jnp.pad(W2, ((0, 0), (0, W48 - N_CLASSES)))
    b1r = b1[None, :]
    b2p = jnp.pad(b2, (0, W48 - N_CLASSES))[None, :]

    degp = _sc_degree(dst_flat)

    y1 = pl.pallas_call(
        _tc1_body,
        grid=(GRID,),
        in_specs=[
            pl.BlockSpec((BLK, D_IN), lambda i: (i, 0)),
            pl.BlockSpec((D_IN, D_HID), lambda i: (0, 0)),
            pl.BlockSpec((D_HID, W48), lambda i: (0, 0)),
            pl.BlockSpec((NW, BLK), lambda i: (0, i)),
        ],
        out_specs=pl.BlockSpec((BLK, W48), lambda i: (i, 0)),
        out_shape=jax.ShapeDtypeStruct((NP, W48), jnp.float32),
    )(xp, W1, w2p, degp)

    accb = _sc_propagate(y1, src, dst)

    y2 = pl.pallas_call(
        _tc2_body,
        grid=(GRID,),
        in_specs=[
            pl.BlockSpec((NC, BLK, W48), lambda i: (0, i, 0)),
            pl.BlockSpec((BLK, W48), lambda i: (i, 0)),
        ],
        out_specs=pl.BlockSpec((BLK, W48), lambda i: (i, 0)),
        out_shape=jax.ShapeDtypeStruct((NP, W48), jnp.float32),
    )(accb, y1)

    accc = _sc_propagate(y2, src, dst)

    out = pl.pallas_call(
        _tc3_body,
        grid=(GRID3,),
        in_specs=[
            pl.BlockSpec((NC, BLK3, W48), lambda i: (0, i, 0)),
            pl.BlockSpec((BLK3, W48), lambda i: (i, 0)),
            pl.BlockSpec((BLK3, W48), lambda i: (i, 0)),
            pl.BlockSpec((D_HID, W48), lambda i: (0, 0)),
            pl.BlockSpec((1, D_HID), lambda i: (0, 0)),
            pl.BlockSpec((1, W48), lambda i: (0, 0)),
        ],
        out_specs=pl.BlockSpec((BLK3, N_CLASSES), lambda i: (i, 0)),
        out_shape=jax.ShapeDtypeStruct((N, N_CLASSES), jnp.float32),
    )(accc, y2, y1, w2p, b1r, b2p)

    return out


# speedup vs baseline: 1.1612x; 1.0877x over previous
import functools

import jax
import jax.numpy as jnp
from jax import lax
from jax.experimental import pallas as pl
from jax.experimental.pallas import tpu as pltpu
from jax.experimental.pallas import tpu_sc as plsc

N = 10000
E = 320000
D_IN = 128
D_HID = 256
N_CLASSES = 40
W48 = 48

NC = 2
NS = 16
NW = NC * NS
NP = 10240
K = 128
EPT = NP
NCHUNK = EPT // K
EPAD = NW * NP - E
RPT = NP // NS

_MESH = plsc.VectorSubcoreMesh(
    core_axis_name="c", subcore_axis_name="s", num_cores=NC, num_subcores=NS
)
_SC_PARAMS = pltpu.CompilerParams(
    needs_layout_passes=False, use_tc_tiling_on_sc=False
)

BLK = 1024
GRID = NP // BLK
BLK3 = 1000
GRID3 = N // BLK3


@functools.partial(
    pl.kernel,
    out_type=jax.ShapeDtypeStruct((NW, NP), jnp.float32),
    mesh=_MESH,
    compiler_params=_SC_PARAMS,
    scratch_types=[
        pltpu.VMEM((EPT,), jnp.int32),
        pltpu.VMEM((NP,), jnp.float32),
    ],
)
def _sc_degree(dst_hbm, out_hbm, dsti_v, deg_v):
    cid = lax.axis_index("c")
    sid = lax.axis_index("s")
    wid = sid * NC + cid
    zeros16 = jnp.zeros((16,), jnp.float32)

    @pl.loop(0, NP // 16)
    def _zero(z):
        deg_v[pl.ds(z * 16, 16)] = zeros16

    pltpu.sync_copy(dst_hbm.at[wid], dsti_v)
    ones16 = jnp.ones((16,), jnp.float32)

    @pl.loop(0, EPT // 16)
    def _count(g):
        idx16 = dsti_v[pl.ds(g * 16, 16)]
        plsc.addupdate_scatter(deg_v, [idx16], ones16)

    pltpu.sync_copy(deg_v, out_hbm.at[wid])


@functools.partial(
    pl.kernel,
    out_type=jax.ShapeDtypeStruct((NC, NP, W48), jnp.float32),
    mesh=_MESH,
    compiler_params=_SC_PARAMS,
    scratch_types=[
        pltpu.MemorySpace.VMEM_SHARED((NP, W48), jnp.float32),
        pltpu.VMEM((NCHUNK, K), jnp.int32),
        pltpu.VMEM((NCHUNK, K), jnp.int32),
        [pltpu.VMEM((K, W48), jnp.float32)] * 8,
        pltpu.VMEM((RPT, W48), jnp.float32),
        [pltpu.SemaphoreType.DMA] * 8,
        [pltpu.SemaphoreType.DMA] * 8,
    ],
)
def _sc_propagate(table_hbm, src_hbm, dst_hbm, out_hbm,
                  accum, srcv, dstv, rows, zbuf, semg, sems):
    cid = lax.axis_index("c")
    sid = lax.axis_index("s")
    wid = sid * NC + cid
    zeros16 = jnp.zeros((16,), jnp.float32)

    @pl.loop(0, RPT)
    def _zrow(r):
        for c3 in range(W48 // 16):
            zbuf[r, pl.ds(c3 * 16, 16)] = zeros16

    pltpu.sync_copy(zbuf, accum.at[pl.ds(sid * RPT, RPT)])

    pltpu.sync_copy(src_hbm.at[wid], srcv)
    pltpu.sync_copy(dst_hbm.at[wid], dstv)
    plsc.subcore_barrier()

    DEPTH = 8
    AHEAD = DEPTH // 2

    def gather(j, u):
        return pltpu.async_copy(table_hbm.at[srcv.at[j]], rows[u], semg[u])

    def scat(j, u):
        return pltpu.async_copy(rows[u], accum.at[dstv.at[j]], sems[u],
                                add=True)

    for u0 in range(AHEAD):
        gather(u0, u0)

    NQ = NCHUNK // DEPTH

    @pl.loop(0, NQ)
    def _edges(q):
        for u in range(DEPTH):
            j = q * DEPTH + u
            pltpu.make_async_copy(table_hbm.at[srcv.at[j]], rows[u],
                                  semg[u]).wait()
            scat(j, u)
            un = (u + AHEAD) % DEPTH

            def wait_scat():
                pltpu.make_async_copy(rows[un], accum.at[dstv.at[j - AHEAD]],
                                      sems[un]).wait()

            if u < AHEAD:
                @pl.when(q > 0)
                def _():
                    wait_scat()

                gather(j + AHEAD, un)
            else:
                wait_scat()

                @pl.when(q < NQ - 1)
                def _():
                    gather(j + AHEAD, un)

    for jt in range(NCHUNK - AHEAD, NCHUNK):
        pltpu.make_async_copy(rows[jt % DEPTH], accum.at[dstv.at[jt]],
                              sems[jt % DEPTH]).wait()
    plsc.subcore_barrier()

    pltpu.sync_copy(accum.at[pl.ds(sid * RPT, RPT)], zbuf)
    pltpu.sync_copy(zbuf, out_hbm.at[cid, pl.ds(sid * RPT, RPT)])


def _colmask(nrows):
    return (lax.broadcasted_iota(jnp.int32, (nrows, W48), 1) == 40).astype(
        jnp.float32
    )


def _col40(a):
    return jnp.sum(jnp.where(_colmask(a.shape[0]) > 0, a, 0.0), axis=1,
                   keepdims=True)


def _tc1_body(x_ref, w1_ref, w2p_ref, degp_ref, y1_ref):
    w12 = jnp.dot(w1_ref[...], w2p_ref[...], preferred_element_type=jnp.float32)
    p = jnp.dot(x_ref[...], w12, preferred_element_type=jnp.float32)
    deg = jnp.sum(degp_ref[...], axis=0)[:, None] + 1.0
    dinv = lax.rsqrt(deg)
    y1_ref[...] = dinv * (p + _colmask(p.shape[0]))


def _tc2_body(acc_ref, y1_ref, y2_ref):
    y1 = y1_ref[...]
    dinv = _col40(y1)
    t = acc_ref[0] + acc_ref[1] + y1
    y2_ref[...] = (dinv * dinv) * t


def _tc3_body(acc_ref, y2_ref, y1_ref, w2p_ref, b1_ref, b2p_ref, out_ref):
    y2 = y2_ref[...]
    dinv = _col40(y1_ref[...])
    svec = _col40(y2) / dinv
    bw = jnp.dot(b1_ref[...], w2p_ref[...], preferred_element_type=jnp.float32)
    o = dinv * (acc_ref[0] + acc_ref[1] + y2)
    o = o + svec * bw + b2p_ref[...]
    out_ref[...] = o[:, :N_CLASSES]


def kernel(x, edge_index, W1, b1, W2, b2):
    padr = N + jnp.arange(EPAD, dtype=jnp.int32) % (NP - N)
    ei = jnp.concatenate([edge_index, jnp.stack([padr, padr])], axis=1)
    src = ei[0].reshape(NW, NCHUNK, K)
    dst = ei[1].reshape(NW, NCHUNK, K)
    dst_flat = ei[1].reshape(NW, EPT)
    xp = jnp.pad(x, ((0, NP - N), (0, 0)))

    w2p = jnp.pad(W2, ((0, 0), (0, W48 - N_CLASSES)))
    b1r = b1[None, :]
    b2p = jnp.pad(b2, (0, W48 - N_CLASSES))[None, :]

    degp = _sc_degree(dst_flat)

    y1 = pl.pallas_call(
        _tc1_body,
        grid=(GRID,),
        in_specs=[
            pl.BlockSpec((BLK, D_IN), lambda i: (i, 0)),
            pl.BlockSpec((D_IN, D_HID), lambda i: (0, 0)),
            pl.BlockSpec((D_HID, W48), lambda i: (0, 0)),
            pl.BlockSpec((NW, BLK), lambda i: (0, i)),
        ],
        out_specs=pl.BlockSpec((BLK, W48), lambda i: (i, 0)),
        out_shape=jax.ShapeDtypeStruct((NP, W48), jnp.float32),
    )(xp, W1, w2p, degp)

    accb = _sc_propagate(y1, src, dst)

    y2 = pl.pallas_call(
        _tc2_body,
        grid=(GRID,),
        in_specs=[
            pl.BlockSpec((NC, BLK, W48), lambda i: (0, i, 0)),
            pl.BlockSpec((BLK, W48), lambda i: (i, 0)),
        ],
        out_specs=pl.BlockSpec((BLK, W48), lambda i: (i, 0)),
        out_shape=jax.ShapeDtypeStruct((NP, W48), jnp.float32),
    )(accb, y1)

    accc = _sc_propagate(y2, src, dst)

    out = pl.pallas_call(
        _tc3_body,
        grid=(GRID3,),
        in_specs=[
            pl.BlockSpec((NC, BLK3, W48), lambda i: (0, i, 0)),
            pl.BlockSpec((BLK3, W48), lambda i: (i, 0)),
            pl.BlockSpec((BLK3, W48), lambda i: (i, 0)),
            pl.BlockSpec((D_HID, W48), lambda i: (0, 0)),
            pl.BlockSpec((1, D_HID), lambda i: (0, 0)),
            pl.BlockSpec((1, W48), lambda i: (0, 0)),
        ],
        out_specs=pl.BlockSpec((BLK3, N_CLASSES), lambda i: (i, 0)),
        out_shape=jax.ShapeDtypeStruct((N, N_CLASSES), jnp.float32),
    )(accc, y2, y1, w2p, b1r, b2p)

    return out
